# Initial kernel scaffold; baseline (speedup 1.0000x reference)
import jax
import jax.numpy as jnp
from jax.experimental import pallas as pl


def kernel(x, edge_index, W1, Wmu, Wlv):
    raise NotImplementedError("write your pallas kernel here")


# speedup vs baseline: 11.6092x; 11.6092x over previous
import functools

import jax
import jax.numpy as jnp
from jax import lax
from jax.experimental import pallas as pl
from jax.experimental.pallas import tpu as pltpu
from jax.experimental.pallas import tpu_sc as plsc


def _enc1(hists, x, W1, RB=1024):
    NWh, N = hists.shape
    F = x.shape[1]
    H = W1.shape[1]

    def body(h_ref, x_ref, w_ref, u_ref, a_ref):
        ones = jnp.ones((NWh, 1), jnp.float32)
        deg = lax.dot_general(h_ref[...], ones, (((0,), (0,)), ((), ()))) - 1.0
        a = lax.rsqrt(deg)
        v = jnp.dot(x_ref[...], w_ref[...], preferred_element_type=jnp.float32)
        u_ref[...] = v * a
        a_ref[...] = a

    return pl.pallas_call(
        body,
        grid=(pl.cdiv(N, RB),),
        in_specs=[
            pl.BlockSpec((NWh, RB), lambda i: (0, i)),
            pl.BlockSpec((RB, F), lambda i: (i, 0)),
            pl.BlockSpec((F, H), lambda i: (0, 0)),
        ],
        out_specs=[
            pl.BlockSpec((RB, H), lambda i: (i, 0)),
            pl.BlockSpec((RB, 1), lambda i: (i, 0)),
        ],
        out_shape=[
            jax.ShapeDtypeStruct((N, H), jnp.float32),
            jax.ShapeDtypeStruct((N, 1), jnp.float32),
        ],
    )(hists, x, W1)


def _enc2(P, u1, a, Wcat, RB=1024):
    N, H = u1.shape
    C2 = Wcat.shape[1]

    def body(p_ref, u_ref, a_ref, w_ref, o_ref):
        s = p_ref[0] + p_ref[1] - u_ref[...]
        h = jnp.maximum(s * a_ref[...], 0.0)
        o_ref[...] = (
            jnp.dot(h, w_ref[...], preferred_element_type=jnp.float32) * a_ref[...]
        )

    return pl.pallas_call(
        body,
        grid=(pl.cdiv(N, RB),),
        in_specs=[
            pl.BlockSpec((2, RB, H), lambda i: (0, i, 0)),
            pl.BlockSpec((RB, H), lambda i: (i, 0)),
            pl.BlockSpec((RB, 1), lambda i: (i, 0)),
            pl.BlockSpec((H, C2), lambda i: (0, 0)),
        ],
        out_specs=pl.BlockSpec((RB, C2), lambda i: (i, 0)),
        out_shape=jax.ShapeDtypeStruct((N, C2), jnp.float32),
    )(P, u1, a, Wcat)


def _decoder(P2, u2, a, L, BM=1024, BN=1024):
    N, C2 = u2.shape

    def body(pi_ref, ui_ref, ai_ref, pj_ref, uj_ref, aj_ref, adj_ref, mu_ref, lv_ref):
        j = pl.program_id(1)
        mi = (pi_ref[0] + pi_ref[1] - ui_ref[...]) * ai_ref[...]
        mj = (pj_ref[0] + pj_ref[1] - uj_ref[...]) * aj_ref[...]
        mui = mi[:, :L]
        muj = mj[:, :L]
        adj_ref[...] = jax.nn.sigmoid(
            lax.dot_general(mui, muj, (((1,), (1,)), ((), ())))
        )

        @pl.when(j == 0)
        def _():
            mu_ref[...] = mui
            lv_ref[...] = mi[:, L:]

    return pl.pallas_call(
        body,
        grid=(pl.cdiv(N, BM), pl.cdiv(N, BN)),
        in_specs=[
            pl.BlockSpec((2, BM, C2), lambda i, j: (0, i, 0)),
            pl.BlockSpec((BM, C2), lambda i, j: (i, 0)),
            pl.BlockSpec((BM, 1), lambda i, j: (i, 0)),
            pl.BlockSpec((2, BN, C2), lambda i, j: (0, j, 0)),
            pl.BlockSpec((BN, C2), lambda i, j: (j, 0)),
            pl.BlockSpec((BN, 1), lambda i, j: (j, 0)),
        ],
        out_specs=[
            pl.BlockSpec((BM, BN), lambda i, j: (i, j)),
            pl.BlockSpec((BM, L), lambda i, j: (i, 0)),
            pl.BlockSpec((BM, L), lambda i, j: (i, 0)),
        ],
        out_shape=[
            jax.ShapeDtypeStruct((N, N), jnp.float32),
            jax.ShapeDtypeStruct((N, L), jnp.float32),
            jax.ShapeDtypeStruct((N, L), jnp.float32),
        ],
    )(P2, u2, a, P2, u2, a)


_NC, _NS = 2, 16
_NW = _NC * _NS


def _deg_hists(dst, N):
    ones = jnp.ones((N, 1), jnp.float32)
    P = _edge_aggregate(ones, dst, dst)
    return P.reshape(_NC, N)


def _edge_aggregate(u, src, dst):
    N, C = u.shape
    E = src.shape[0]
    B = 80
    assert E % (_NW * B) == 0
    EPW = E // _NW
    NCHUNK = EPW // B
    RPT = ((-(-N // _NS) + 7) // 8) * 8
    LAST = N - RPT * (_NS - 1)
    assert 0 < LAST <= RPT and RPT % 8 == 0
    mesh = plsc.VectorSubcoreMesh(core_axis_name="c", subcore_axis_name="s")

    @functools.partial(
        pl.kernel,
        out_type=jax.ShapeDtypeStruct((_NC, N, C), jnp.float32),
        mesh=mesh,
        compiler_params=pltpu.CompilerParams(use_tc_tiling_on_sc=False),
        scratch_types=[
            pltpu.VMEM_SHARED((N, C), jnp.float32),
            pltpu.VMEM((B,), jnp.int32),
            pltpu.VMEM((B,), jnp.int32),
            pltpu.VMEM((B, C), jnp.float32),
            pltpu.VMEM((RPT, C), jnp.float32),
            pltpu.SemaphoreType.DMA,
        ],
    )
    def k(u_hbm, src_hbm, dst_hbm, out_hbm, acc, sidx, didx, rows, stage, sem):
        c = lax.axis_index("c")
        s = lax.axis_index("s")
        wid = s * _NC + c
        r0 = s * RPT
        @pl.when(s < _NS - 1)
        def _():
            pltpu.sync_copy(u_hbm.at[pl.ds(r0, RPT)], stage)
            pltpu.sync_copy(stage, acc.at[pl.ds(r0, RPT)])

        @pl.when(s == _NS - 1)
        def _():
            pltpu.sync_copy(u_hbm.at[pl.ds(r0, LAST)], stage.at[pl.ds(0, LAST)])
            pltpu.sync_copy(stage.at[pl.ds(0, LAST)], acc.at[pl.ds(r0, LAST)])

        plsc.subcore_barrier()
        e0 = wid * EPW

        @pl.loop(0, NCHUNK)
        def _(i):
            base = e0 + i * B
            pltpu.sync_copy(src_hbm.at[pl.ds(base, B)], sidx)
            pltpu.sync_copy(dst_hbm.at[pl.ds(base, B)], didx)
            pltpu.async_copy(u_hbm.at[sidx], rows, sem).wait()
            pltpu.sync_copy(rows, acc.at[didx], add=True)

        plsc.subcore_barrier()

        @pl.when(s < _NS - 1)
        def _():
            pltpu.sync_copy(acc.at[pl.ds(r0, RPT)], stage)
            pltpu.sync_copy(stage, out_hbm.at[c, pl.ds(r0, RPT)])

        @pl.when(s == _NS - 1)
        def _():
            pltpu.sync_copy(acc.at[pl.ds(r0, LAST)], stage.at[pl.ds(0, LAST)])
            pltpu.sync_copy(stage.at[pl.ds(0, LAST)], out_hbm.at[c, pl.ds(r0, LAST)])

    return k(u, src, dst)


def kernel(x, edge_index, W1, Wmu, Wlv):
    N = x.shape[0]
    L = Wmu.shape[1]
    src, dst = edge_index[0], edge_index[1]
    Wcat = jnp.concatenate([Wmu, Wlv], axis=1)

    hists = _deg_hists(dst, N)
    u1, a = _enc1(hists, x, W1)
    P1 = _edge_aggregate(u1, src, dst)
    u2 = _enc2(P1, u1, a, Wcat)
    P2 = _edge_aggregate(u2, src, dst)
    adj, mu, logvar = _decoder(P2, u2, a, L)
    return (adj, mu, logvar)


# speedup vs baseline: 27.3057x; 2.3521x over previous
import functools

import jax
import jax.numpy as jnp
from jax import lax
from jax.experimental import pallas as pl
from jax.experimental.pallas import tpu as pltpu
from jax.experimental.pallas import tpu_sc as plsc


def _enc1(hists, x, W1, RB=1024):
    NWh, N = hists.shape
    F = x.shape[1]
    H = W1.shape[1]

    def body(h_ref, x_ref, w_ref, u_ref, a_ref):
        ones = jnp.ones((NWh, 1), jnp.float32)
        deg = lax.dot_general(h_ref[...], ones, (((0,), (0,)), ((), ()))) + 1.0
        a = lax.rsqrt(deg)
        v = jnp.dot(x_ref[...], w_ref[...], preferred_element_type=jnp.float32)
        u_ref[...] = v * a
        a_ref[...] = a

    return pl.pallas_call(
        body,
        grid=(pl.cdiv(N, RB),),
        in_specs=[
            pl.BlockSpec((NWh, RB), lambda i: (0, i)),
            pl.BlockSpec((RB, F), lambda i: (i, 0)),
            pl.BlockSpec((F, H), lambda i: (0, 0)),
        ],
        out_specs=[
            pl.BlockSpec((RB, H), lambda i: (i, 0)),
            pl.BlockSpec((RB, 1), lambda i: (i, 0)),
        ],
        out_shape=[
            jax.ShapeDtypeStruct((N, H), jnp.float32),
            jax.ShapeDtypeStruct((N, 1), jnp.float32),
        ],
    )(hists, x, W1)


def _enc2(P, u1, a, Wcat, RB=1024):
    N, H = u1.shape
    C2 = Wcat.shape[1]

    def body(p_ref, u_ref, a_ref, w_ref, o_ref):
        s = p_ref[0] + p_ref[1] - u_ref[...]
        h = jnp.maximum(s * a_ref[...], 0.0)
        o_ref[...] = (
            jnp.dot(h, w_ref[...], preferred_element_type=jnp.float32) * a_ref[...]
        )

    return pl.pallas_call(
        body,
        grid=(pl.cdiv(N, RB),),
        in_specs=[
            pl.BlockSpec((2, RB, H), lambda i: (0, i, 0)),
            pl.BlockSpec((RB, H), lambda i: (i, 0)),
            pl.BlockSpec((RB, 1), lambda i: (i, 0)),
            pl.BlockSpec((H, C2), lambda i: (0, 0)),
        ],
        out_specs=pl.BlockSpec((RB, C2), lambda i: (i, 0)),
        out_shape=jax.ShapeDtypeStruct((N, C2), jnp.float32),
    )(P, u1, a, Wcat)


def _decoder(P2, u2, a, L, BM=1024, BN=1024):
    N, C2 = u2.shape

    def body(pi_ref, ui_ref, ai_ref, pj_ref, uj_ref, aj_ref, adj_ref, mu_ref, lv_ref):
        j = pl.program_id(1)
        mi = (pi_ref[0] + pi_ref[1] - ui_ref[...]) * ai_ref[...]
        mj = (pj_ref[0] + pj_ref[1] - uj_ref[...]) * aj_ref[...]
        mui = mi[:, :L]
        muj = mj[:, :L]
        adj_ref[...] = jax.nn.sigmoid(
            lax.dot_general(mui, muj, (((1,), (1,)), ((), ())))
        )

        @pl.when(j == 0)
        def _():
            mu_ref[...] = mui
            lv_ref[...] = mi[:, L:]

    return pl.pallas_call(
        body,
        grid=(pl.cdiv(N, BM), pl.cdiv(N, BN)),
        in_specs=[
            pl.BlockSpec((2, BM, C2), lambda i, j: (0, i, 0)),
            pl.BlockSpec((BM, C2), lambda i, j: (i, 0)),
            pl.BlockSpec((BM, 1), lambda i, j: (i, 0)),
            pl.BlockSpec((2, BN, C2), lambda i, j: (0, j, 0)),
            pl.BlockSpec((BN, C2), lambda i, j: (j, 0)),
            pl.BlockSpec((BN, 1), lambda i, j: (j, 0)),
        ],
        out_specs=[
            pl.BlockSpec((BM, BN), lambda i, j: (i, j)),
            pl.BlockSpec((BM, L), lambda i, j: (i, 0)),
            pl.BlockSpec((BM, L), lambda i, j: (i, 0)),
        ],
        out_shape=[
            jax.ShapeDtypeStruct((N, N), jnp.float32),
            jax.ShapeDtypeStruct((N, L), jnp.float32),
            jax.ShapeDtypeStruct((N, L), jnp.float32),
        ],
    )(P2, u2, a, P2, u2, a)


_NC, _NS = 2, 16
_NW = _NC * _NS


def _deg_hists(dstT, N):
    T, B = dstT.shape
    NBUF = 4
    assert T % _NW == 0
    TPW = T // _NW
    assert TPW % NBUF == 0 and B <= 128
    RPT = ((-(-N // _NS) + 7) // 8) * 8
    LAST = N - RPT * (_NS - 1)
    SBUF = ((RPT + 15) // 16) * 16
    assert 0 < LAST <= RPT and RPT % 8 == 0
    mesh = plsc.VectorSubcoreMesh(core_axis_name="c", subcore_axis_name="s")

    @functools.partial(
        pl.kernel,
        out_type=jax.ShapeDtypeStruct((_NC, N), jnp.float32),
        mesh=mesh,
        compiler_params=pltpu.CompilerParams(use_tc_tiling_on_sc=False),
        scratch_types=[
            pltpu.VMEM_SHARED((N,), jnp.float32),
            pltpu.VMEM((TPW, B), jnp.int32),
            pltpu.VMEM((128,), jnp.float32),
            pltpu.VMEM((SBUF,), jnp.float32),
        ]
        + [pltpu.SemaphoreType.DMA] * NBUF,
    )
    def k(dstT_hbm, out_hbm, hist, didx, onez, stage, *ss):
        c = lax.axis_index("c")
        s = lax.axis_index("s")
        wid = s * _NC + c
        r0 = s * RPT
        ones16 = jnp.ones((16,), jnp.float32)
        zeros16 = jnp.zeros((16,), jnp.float32)
        for i in range(8):
            onez[pl.ds(16 * i, 16)] = ones16

        @pl.loop(0, SBUF // 16)
        def _(i):
            stage[pl.ds(16 * i, 16)] = zeros16

        @pl.when(s < _NS - 1)
        def _():
            pltpu.sync_copy(stage.at[pl.ds(0, RPT)], hist.at[pl.ds(r0, RPT)])

        @pl.when(s == _NS - 1)
        def _():
            pltpu.sync_copy(stage.at[pl.ds(0, LAST)], hist.at[pl.ds(r0, LAST)])

        plsc.subcore_barrier()
        t0 = wid * TPW
        pltpu.sync_copy(dstT_hbm.at[pl.ds(t0, TPW)], didx)

        @pl.loop(0, TPW // NBUF)
        def _(it):
            i = it * NBUF
            for b in range(NBUF):
                pltpu.async_copy(
                    onez.at[pl.ds(0, B)], hist.at[didx.at[i + b]], ss[b], add=True
                )
            for b in range(NBUF):
                pltpu.make_async_copy(
                    onez.at[pl.ds(0, B)], hist.at[didx.at[i + b]], ss[b]
                ).wait()

        plsc.subcore_barrier()

        @pl.when(s < _NS - 1)
        def _():
            pltpu.sync_copy(hist.at[pl.ds(r0, RPT)], stage.at[pl.ds(0, RPT)])
            pltpu.sync_copy(stage.at[pl.ds(0, RPT)], out_hbm.at[c, pl.ds(r0, RPT)])

        @pl.when(s == _NS - 1)
        def _():
            pltpu.sync_copy(hist.at[pl.ds(r0, LAST)], stage.at[pl.ds(0, LAST)])
            pltpu.sync_copy(stage.at[pl.ds(0, LAST)], out_hbm.at[c, pl.ds(r0, LAST)])

    return k(dstT)


_B = 125


def _edge_aggregate(u, srcT, dstT):
    N, C = u.shape
    T, B = srcT.shape
    NBUF = 4
    assert T % _NW == 0
    TPW = T // _NW
    assert TPW % NBUF == 0 and TPW >= 2 * NBUF and B <= 128
    RPT = ((-(-N // _NS) + 7) // 8) * 8
    LAST = N - RPT * (_NS - 1)
    assert 0 < LAST <= RPT and RPT % 8 == 0
    mesh = plsc.VectorSubcoreMesh(core_axis_name="c", subcore_axis_name="s")

    @functools.partial(
        pl.kernel,
        out_type=jax.ShapeDtypeStruct((_NC, N, C), jnp.float32),
        mesh=mesh,
        compiler_params=pltpu.CompilerParams(use_tc_tiling_on_sc=False),
        scratch_types=[
            pltpu.VMEM_SHARED((N, C), jnp.float32),
            pltpu.VMEM((TPW, B), jnp.int32),
            pltpu.VMEM((TPW, B), jnp.int32),
            pltpu.VMEM((NBUF, B, C), jnp.float32),
        ]
        + [pltpu.SemaphoreType.DMA] * (2 * NBUF),
    )
    def k(u_hbm, srcT_hbm, dstT_hbm, out_hbm, acc, sidx, didx, rows, *sems):
        gs, ss = sems[:NBUF], sems[NBUF:]
        c = lax.axis_index("c")
        s = lax.axis_index("s")
        wid = s * _NC + c
        r0 = s * RPT
        @pl.when(s < _NS - 1)
        def _():
            pltpu.sync_copy(u_hbm.at[pl.ds(r0, RPT)], acc.at[pl.ds(r0, RPT)])

        @pl.when(s == _NS - 1)
        def _():
            pltpu.sync_copy(u_hbm.at[pl.ds(r0, LAST)], acc.at[pl.ds(r0, LAST)])

        plsc.subcore_barrier()
        t0 = wid * TPW
        pltpu.sync_copy(srcT_hbm.at[pl.ds(t0, TPW)], sidx)
        pltpu.sync_copy(dstT_hbm.at[pl.ds(t0, TPW)], didx)
        for b in range(NBUF):
            pltpu.async_copy(u_hbm.at[sidx.at[b]], rows.at[b], gs[b])

        @pl.loop(0, (TPW - NBUF) // NBUF)
        def _(it):
            i = it * NBUF
            for b in range(NBUF):
                pltpu.make_async_copy(
                    u_hbm.at[sidx.at[i + b]], rows.at[b], gs[b]
                ).wait()
                pltpu.async_copy(rows.at[b], acc.at[didx.at[i + b]], ss[b], add=True)
            for b in range(NBUF):
                pltpu.make_async_copy(
                    rows.at[b], acc.at[didx.at[i + b]], ss[b]
                ).wait()
                pltpu.async_copy(u_hbm.at[sidx.at[i + NBUF + b]], rows.at[b], gs[b])

        ilast = TPW - NBUF
        for b in range(NBUF):
            pltpu.make_async_copy(
                u_hbm.at[sidx.at[ilast + b]], rows.at[b], gs[b]
            ).wait()
            pltpu.async_copy(rows.at[b], acc.at[didx.at[ilast + b]], ss[b], add=True)
        for b in range(NBUF):
            pltpu.make_async_copy(
                rows.at[b], acc.at[didx.at[ilast + b]], ss[b]
            ).wait()

        plsc.subcore_barrier()

        @pl.when(s < _NS - 1)
        def _():
            pltpu.sync_copy(acc.at[pl.ds(r0, RPT)], out_hbm.at[c, pl.ds(r0, RPT)])

        @pl.when(s == _NS - 1)
        def _():
            pltpu.sync_copy(acc.at[pl.ds(r0, LAST)], out_hbm.at[c, pl.ds(r0, LAST)])

    return k(u, srcT, dstT)


def kernel(x, edge_index, W1, Wmu, Wlv):
    N = x.shape[0]
    E = edge_index.shape[1]
    L = Wmu.shape[1]
    srcT = edge_index[0].reshape(E // _B, _B)
    dstT = edge_index[1].reshape(E // _B, _B)
    Wcat = jnp.concatenate([Wmu, Wlv], axis=1)

    hists = _deg_hists(dstT, N)
    u1, a = _enc1(hists, x, W1)
    P1 = _edge_aggregate(u1, srcT, dstT)
    u2 = _enc2(P1, u1, a, Wcat)
    P2 = _edge_aggregate(u2, srcT, dstT)
    adj, mu, logvar = _decoder(P2, u2, a, L)
    return (adj, mu, logvar)


# speedup vs baseline: 28.3035x; 1.0365x over previous
import functools

import jax
import jax.numpy as jnp
from jax import lax
from jax.experimental import pallas as pl
from jax.experimental.pallas import tpu as pltpu
from jax.experimental.pallas import tpu_sc as plsc


def _enc1(hists, x, W1, RB=1024):
    NWh, N = hists.shape
    F = x.shape[1]
    H = W1.shape[1]

    def body(h_ref, x_ref, w_ref, u_ref, a_ref):
        ones = jnp.ones((NWh, 1), jnp.float32)
        deg = lax.dot_general(h_ref[...], ones, (((0,), (0,)), ((), ()))) + 1.0
        a = lax.rsqrt(deg)
        v = jnp.dot(x_ref[...], w_ref[...], preferred_element_type=jnp.float32)
        u_ref[...] = v * a
        a_ref[...] = a

    return pl.pallas_call(
        body,
        grid=(pl.cdiv(N, RB),),
        in_specs=[
            pl.BlockSpec((NWh, RB), lambda i: (0, i)),
            pl.BlockSpec((RB, F), lambda i: (i, 0)),
            pl.BlockSpec((F, H), lambda i: (0, 0)),
        ],
        out_specs=[
            pl.BlockSpec((RB, H), lambda i: (i, 0)),
            pl.BlockSpec((RB, 1), lambda i: (i, 0)),
        ],
        out_shape=[
            jax.ShapeDtypeStruct((N, H), jnp.float32),
            jax.ShapeDtypeStruct((N, 1), jnp.float32),
        ],
    )(hists, x, W1)


def _enc2(P, u1, a, Wcat, RB=1024):
    N, H = u1.shape
    C2 = Wcat.shape[1]

    def body(p_ref, u_ref, a_ref, w_ref, o_ref):
        s = p_ref[0] + p_ref[1] - u_ref[...]
        h = jnp.maximum(s * a_ref[...], 0.0)
        o_ref[...] = (
            jnp.dot(h, w_ref[...], preferred_element_type=jnp.float32) * a_ref[...]
        )

    return pl.pallas_call(
        body,
        grid=(pl.cdiv(N, RB),),
        in_specs=[
            pl.BlockSpec((2, RB, H), lambda i: (0, i, 0)),
            pl.BlockSpec((RB, H), lambda i: (i, 0)),
            pl.BlockSpec((RB, 1), lambda i: (i, 0)),
            pl.BlockSpec((H, C2), lambda i: (0, 0)),
        ],
        out_specs=pl.BlockSpec((RB, C2), lambda i: (i, 0)),
        out_shape=jax.ShapeDtypeStruct((N, C2), jnp.float32),
    )(P, u1, a, Wcat)


def _decoder(P2, u2, a, L, BM=1024, BN=1024):
    N, C2 = u2.shape

    def body(pi_ref, ui_ref, ai_ref, pj_ref, uj_ref, aj_ref, adj_ref, mu_ref, lv_ref):
        j = pl.program_id(1)
        mi = (pi_ref[0] + pi_ref[1] - ui_ref[...]) * ai_ref[...]
        mj = (pj_ref[0] + pj_ref[1] - uj_ref[...]) * aj_ref[...]
        mui = mi[:, :L]
        muj = mj[:, :L]
        d = lax.dot_general(mui, muj, (((1,), (1,)), ((), ())))
        adj_ref[...] = jnp.tanh(d * 0.5) * 0.5 + 0.5

        @pl.when(j == 0)
        def _():
            mu_ref[...] = mui
            lv_ref[...] = mi[:, L:]

    return pl.pallas_call(
        body,
        grid=(pl.cdiv(N, BM), pl.cdiv(N, BN)),
        in_specs=[
            pl.BlockSpec((2, BM, C2), lambda i, j: (0, i, 0)),
            pl.BlockSpec((BM, C2), lambda i, j: (i, 0)),
            pl.BlockSpec((BM, 1), lambda i, j: (i, 0)),
            pl.BlockSpec((2, BN, C2), lambda i, j: (0, j, 0)),
            pl.BlockSpec((BN, C2), lambda i, j: (j, 0)),
            pl.BlockSpec((BN, 1), lambda i, j: (j, 0)),
        ],
        out_specs=[
            pl.BlockSpec((BM, BN), lambda i, j: (i, j)),
            pl.BlockSpec((BM, L), lambda i, j: (i, 0)),
            pl.BlockSpec((BM, L), lambda i, j: (i, 0)),
        ],
        out_shape=[
            jax.ShapeDtypeStruct((N, N), jnp.float32),
            jax.ShapeDtypeStruct((N, L), jnp.float32),
            jax.ShapeDtypeStruct((N, L), jnp.float32),
        ],
    )(P2, u2, a, P2, u2, a)


_NC, _NS = 2, 16
_NW = _NC * _NS


def _deg_hists(dstT, N):
    T, B = dstT.shape
    NBUF = 4
    assert T % _NW == 0
    TPW = T // _NW
    assert TPW % NBUF == 0 and B <= 128
    RPT = ((-(-N // _NS) + 7) // 8) * 8
    LAST = N - RPT * (_NS - 1)
    SBUF = ((RPT + 15) // 16) * 16
    assert 0 < LAST <= RPT and RPT % 8 == 0
    mesh = plsc.VectorSubcoreMesh(core_axis_name="c", subcore_axis_name="s")

    @functools.partial(
        pl.kernel,
        out_type=jax.ShapeDtypeStruct((_NC, N), jnp.float32),
        mesh=mesh,
        compiler_params=pltpu.CompilerParams(use_tc_tiling_on_sc=False),
        scratch_types=[
            pltpu.VMEM_SHARED((N,), jnp.float32),
            pltpu.VMEM((TPW, B), jnp.int32),
            pltpu.VMEM((128,), jnp.float32),
            pltpu.VMEM((SBUF,), jnp.float32),
        ]
        + [pltpu.SemaphoreType.DMA] * NBUF,
    )
    def k(dstT_hbm, out_hbm, hist, didx, onez, stage, *ss):
        c = lax.axis_index("c")
        s = lax.axis_index("s")
        wid = s * _NC + c
        r0 = s * RPT
        ones16 = jnp.ones((16,), jnp.float32)
        zeros16 = jnp.zeros((16,), jnp.float32)
        for i in range(8):
            onez[pl.ds(16 * i, 16)] = ones16

        @pl.loop(0, SBUF // 16)
        def _(i):
            stage[pl.ds(16 * i, 16)] = zeros16

        @pl.when(s < _NS - 1)
        def _():
            pltpu.sync_copy(stage.at[pl.ds(0, RPT)], hist.at[pl.ds(r0, RPT)])

        @pl.when(s == _NS - 1)
        def _():
            pltpu.sync_copy(stage.at[pl.ds(0, LAST)], hist.at[pl.ds(r0, LAST)])

        plsc.subcore_barrier()
        t0 = wid * TPW
        pltpu.sync_copy(dstT_hbm.at[pl.ds(t0, TPW)], didx)

        @pl.loop(0, TPW // NBUF)
        def _(it):
            i = it * NBUF
            for b in range(NBUF):
                pltpu.async_copy(
                    onez.at[pl.ds(0, B)], hist.at[didx.at[i + b]], ss[b], add=True
                )
            for b in range(NBUF):
                pltpu.make_async_copy(
                    onez.at[pl.ds(0, B)], hist.at[didx.at[i + b]], ss[b]
                ).wait()

        plsc.subcore_barrier()

        @pl.when(s < _NS - 1)
        def _():
            pltpu.sync_copy(hist.at[pl.ds(r0, RPT)], stage.at[pl.ds(0, RPT)])
            pltpu.sync_copy(stage.at[pl.ds(0, RPT)], out_hbm.at[c, pl.ds(r0, RPT)])

        @pl.when(s == _NS - 1)
        def _():
            pltpu.sync_copy(hist.at[pl.ds(r0, LAST)], stage.at[pl.ds(0, LAST)])
            pltpu.sync_copy(stage.at[pl.ds(0, LAST)], out_hbm.at[c, pl.ds(r0, LAST)])

    return k(dstT)


_B = 125


def _edge_aggregate(u, srcT, dstT):
    N, C = u.shape
    T, B = srcT.shape
    NBUF = 4
    assert T % _NW == 0
    TPW = T // _NW
    assert TPW % NBUF == 0 and TPW >= 2 * NBUF and B <= 128
    RPT = ((-(-N // _NS) + 7) // 8) * 8
    LAST = N - RPT * (_NS - 1)
    assert 0 < LAST <= RPT and RPT % 8 == 0
    mesh = plsc.VectorSubcoreMesh(core_axis_name="c", subcore_axis_name="s")

    @functools.partial(
        pl.kernel,
        out_type=jax.ShapeDtypeStruct((_NC, N, C), jnp.float32),
        mesh=mesh,
        compiler_params=pltpu.CompilerParams(use_tc_tiling_on_sc=False),
        scratch_types=[
            pltpu.VMEM_SHARED((N, C), jnp.float32),
            pltpu.VMEM((TPW, B), jnp.int32),
            pltpu.VMEM((TPW, B), jnp.int32),
            pltpu.VMEM((NBUF, B, C), jnp.float32),
        ]
        + [pltpu.SemaphoreType.DMA] * (2 * NBUF),
    )
    def k(u_hbm, srcT_hbm, dstT_hbm, out_hbm, acc, sidx, didx, rows, *sems):
        gs, ss = sems[:NBUF], sems[NBUF:]
        c = lax.axis_index("c")
        s = lax.axis_index("s")
        wid = s * _NC + c
        r0 = s * RPT
        @pl.when(s < _NS - 1)
        def _():
            pltpu.sync_copy(u_hbm.at[pl.ds(r0, RPT)], acc.at[pl.ds(r0, RPT)])

        @pl.when(s == _NS - 1)
        def _():
            pltpu.sync_copy(u_hbm.at[pl.ds(r0, LAST)], acc.at[pl.ds(r0, LAST)])

        plsc.subcore_barrier()
        t0 = wid * TPW
        pltpu.sync_copy(srcT_hbm.at[pl.ds(t0, TPW)], sidx)
        pltpu.sync_copy(dstT_hbm.at[pl.ds(t0, TPW)], didx)
        for b in range(NBUF):
            pltpu.async_copy(u_hbm.at[sidx.at[b]], rows.at[b], gs[b])

        @pl.loop(0, (TPW - NBUF) // NBUF)
        def _(it):
            i = it * NBUF
            for b in range(NBUF):
                pltpu.make_async_copy(
                    u_hbm.at[sidx.at[i + b]], rows.at[b], gs[b]
                ).wait()
                pltpu.async_copy(rows.at[b], acc.at[didx.at[i + b]], ss[b], add=True)
            for b in range(NBUF):
                pltpu.make_async_copy(
                    rows.at[b], acc.at[didx.at[i + b]], ss[b]
                ).wait()
                pltpu.async_copy(u_hbm.at[sidx.at[i + NBUF + b]], rows.at[b], gs[b])

        ilast = TPW - NBUF
        for b in range(NBUF):
            pltpu.make_async_copy(
                u_hbm.at[sidx.at[ilast + b]], rows.at[b], gs[b]
            ).wait()
            pltpu.async_copy(rows.at[b], acc.at[didx.at[ilast + b]], ss[b], add=True)
        for b in range(NBUF):
            pltpu.make_async_copy(
                rows.at[b], acc.at[didx.at[ilast + b]], ss[b]
            ).wait()

        plsc.subcore_barrier()

        @pl.when(s < _NS - 1)
        def _():
            pltpu.sync_copy(acc.at[pl.ds(r0, RPT)], out_hbm.at[c, pl.ds(r0, RPT)])

        @pl.when(s == _NS - 1)
        def _():
            pltpu.sync_copy(acc.at[pl.ds(r0, LAST)], out_hbm.at[c, pl.ds(r0, LAST)])

    return k(u, srcT, dstT)


def kernel(x, edge_index, W1, Wmu, Wlv):
    N = x.shape[0]
    E = edge_index.shape[1]
    L = Wmu.shape[1]
    srcT = edge_index[0].reshape(E // _B, _B)
    dstT = edge_index[1].reshape(E // _B, _B)
    Wcat = jnp.concatenate([Wmu, Wlv], axis=1)

    hists = _deg_hists(dstT, N)
    u1, a = _enc1(hists, x, W1)
    P1 = _edge_aggregate(u1, srcT, dstT)
    u2 = _enc2(P1, u1, a, Wcat)
    P2 = _edge_aggregate(u2, srcT, dstT)
    adj, mu, logvar = _decoder(P2, u2, a, L)
    return (adj, mu, logvar)


# speedup vs baseline: 32.2076x; 1.1379x over previous
import functools

import jax
import jax.numpy as jnp
from jax import lax
from jax.experimental import pallas as pl
from jax.experimental.pallas import tpu as pltpu
from jax.experimental.pallas import tpu_sc as plsc


def _enc1(hists, x, W1, RB=1024):
    NWh, N = hists.shape
    F = x.shape[1]
    H = W1.shape[1]

    def body(h_ref, x_ref, w_ref, u_ref, a_ref):
        ones = jnp.ones((NWh, 1), jnp.float32)
        deg = lax.dot_general(h_ref[...], ones, (((0,), (0,)), ((), ()))) + 1.0
        a = lax.rsqrt(deg)
        v = jnp.dot(x_ref[...], w_ref[...], preferred_element_type=jnp.float32)
        u_ref[...] = v * a
        a_ref[...] = a

    return pl.pallas_call(
        body,
        grid=(pl.cdiv(N, RB),),
        in_specs=[
            pl.BlockSpec((NWh, RB), lambda i: (0, i)),
            pl.BlockSpec((RB, F), lambda i: (i, 0)),
            pl.BlockSpec((F, H), lambda i: (0, 0)),
        ],
        out_specs=[
            pl.BlockSpec((RB, H), lambda i: (i, 0)),
            pl.BlockSpec((RB, 1), lambda i: (i, 0)),
        ],
        out_shape=[
            jax.ShapeDtypeStruct((N, H), jnp.float32),
            jax.ShapeDtypeStruct((N, 1), jnp.float32),
        ],
    )(hists, x, W1)


def _enc2(P, u1, a, Wcat, RB=1024):
    N, H = u1.shape
    C2 = Wcat.shape[1]

    def body(p_ref, u_ref, a_ref, w_ref, o_ref):
        s = p_ref[0] + p_ref[1] - u_ref[...]
        h = jnp.maximum(s * a_ref[...], 0.0)
        o_ref[...] = (
            jnp.dot(h, w_ref[...], preferred_element_type=jnp.float32) * a_ref[...]
        )

    return pl.pallas_call(
        body,
        grid=(pl.cdiv(N, RB),),
        in_specs=[
            pl.BlockSpec((2, RB, H), lambda i: (0, i, 0)),
            pl.BlockSpec((RB, H), lambda i: (i, 0)),
            pl.BlockSpec((RB, 1), lambda i: (i, 0)),
            pl.BlockSpec((H, C2), lambda i: (0, 0)),
        ],
        out_specs=pl.BlockSpec((RB, C2), lambda i: (i, 0)),
        out_shape=jax.ShapeDtypeStruct((N, C2), jnp.float32),
    )(P, u1, a, Wcat)


def _decoder(P2, u2, a, L, BM=2048, BN=1024):
    N, C2 = u2.shape

    def body(pi_ref, ui_ref, ai_ref, pj_ref, uj_ref, aj_ref, adj_ref, mu_ref, lv_ref):
        j = pl.program_id(1)
        mi = (pi_ref[0] + pi_ref[1] - ui_ref[...]) * ai_ref[...]
        mj = (pj_ref[0] + pj_ref[1] - uj_ref[...]) * aj_ref[...]
        mui = mi[:, :L]
        muj = mj[:, :L]
        d = lax.dot_general(mui, muj, (((1,), (1,)), ((), ())))
        adj_ref[...] = jnp.tanh(d * 0.5) * 0.5 + 0.5

        @pl.when(j == 0)
        def _():
            mu_ref[...] = mui
            lv_ref[...] = mi[:, L:]

    return pl.pallas_call(
        body,
        grid=(pl.cdiv(N, BM), pl.cdiv(N, BN)),
        in_specs=[
            pl.BlockSpec((2, BM, C2), lambda i, j: (0, i, 0)),
            pl.BlockSpec((BM, C2), lambda i, j: (i, 0)),
            pl.BlockSpec((BM, 1), lambda i, j: (i, 0)),
            pl.BlockSpec((2, BN, C2), lambda i, j: (0, j, 0)),
            pl.BlockSpec((BN, C2), lambda i, j: (j, 0)),
            pl.BlockSpec((BN, 1), lambda i, j: (j, 0)),
        ],
        out_specs=[
            pl.BlockSpec((BM, BN), lambda i, j: (i, j)),
            pl.BlockSpec((BM, L), lambda i, j: (i, 0)),
            pl.BlockSpec((BM, L), lambda i, j: (i, 0)),
        ],
        out_shape=[
            jax.ShapeDtypeStruct((N, N), jnp.float32),
            jax.ShapeDtypeStruct((N, L), jnp.float32),
            jax.ShapeDtypeStruct((N, L), jnp.float32),
        ],
    )(P2, u2, a, P2, u2, a)


_NC, _NS = 2, 16
_NW = _NC * _NS


def _deg_hists(dstT, N):
    T, B = dstT.shape
    NBUF = 4
    assert T % _NW == 0
    TPW = T // _NW
    assert TPW % NBUF == 0 and B <= 128
    RPT = ((-(-N // _NS) + 7) // 8) * 8
    LAST = N - RPT * (_NS - 1)
    SBUF = ((RPT + 15) // 16) * 16
    assert 0 < LAST <= RPT and RPT % 8 == 0
    mesh = plsc.VectorSubcoreMesh(core_axis_name="c", subcore_axis_name="s")

    @functools.partial(
        pl.kernel,
        out_type=jax.ShapeDtypeStruct((_NC, N), jnp.float32),
        mesh=mesh,
        compiler_params=pltpu.CompilerParams(use_tc_tiling_on_sc=False),
        scratch_types=[
            pltpu.VMEM_SHARED((N,), jnp.float32),
            pltpu.VMEM((TPW, B), jnp.int32),
            pltpu.VMEM((128,), jnp.float32),
            pltpu.VMEM((SBUF,), jnp.float32),
        ]
        + [pltpu.SemaphoreType.DMA] * NBUF,
    )
    def k(dstT_hbm, out_hbm, hist, didx, onez, stage, *ss):
        c = lax.axis_index("c")
        s = lax.axis_index("s")
        wid = s * _NC + c
        r0 = s * RPT
        ones16 = jnp.ones((16,), jnp.float32)
        zeros16 = jnp.zeros((16,), jnp.float32)
        for i in range(8):
            onez[pl.ds(16 * i, 16)] = ones16

        @pl.loop(0, SBUF // 16)
        def _(i):
            stage[pl.ds(16 * i, 16)] = zeros16

        @pl.when(s < _NS - 1)
        def _():
            pltpu.sync_copy(stage.at[pl.ds(0, RPT)], hist.at[pl.ds(r0, RPT)])

        @pl.when(s == _NS - 1)
        def _():
            pltpu.sync_copy(stage.at[pl.ds(0, LAST)], hist.at[pl.ds(r0, LAST)])

        plsc.subcore_barrier()
        t0 = wid * TPW
        pltpu.sync_copy(dstT_hbm.at[pl.ds(t0, TPW)], didx)

        @pl.loop(0, TPW // NBUF)
        def _(it):
            i = it * NBUF
            for b in range(NBUF):
                pltpu.async_copy(
                    onez.at[pl.ds(0, B)], hist.at[didx.at[i + b]], ss[b], add=True
                )
            for b in range(NBUF):
                pltpu.make_async_copy(
                    onez.at[pl.ds(0, B)], hist.at[didx.at[i + b]], ss[b]
                ).wait()

        plsc.subcore_barrier()

        @pl.when(s < _NS - 1)
        def _():
            pltpu.sync_copy(hist.at[pl.ds(r0, RPT)], stage.at[pl.ds(0, RPT)])
            pltpu.sync_copy(stage.at[pl.ds(0, RPT)], out_hbm.at[c, pl.ds(r0, RPT)])

        @pl.when(s == _NS - 1)
        def _():
            pltpu.sync_copy(hist.at[pl.ds(r0, LAST)], stage.at[pl.ds(0, LAST)])
            pltpu.sync_copy(stage.at[pl.ds(0, LAST)], out_hbm.at[c, pl.ds(r0, LAST)])

    return k(dstT)


_B = 125


def _edge_aggregate(u, srcT, dstT):
    N, C = u.shape
    T, B = srcT.shape
    NBUF = 4
    assert T % _NW == 0
    TPW = T // _NW
    assert TPW % NBUF == 0 and TPW >= 2 * NBUF and B <= 128
    RPT = ((-(-N // _NS) + 7) // 8) * 8
    LAST = N - RPT * (_NS - 1)
    assert 0 < LAST <= RPT and RPT % 8 == 0
    mesh = plsc.VectorSubcoreMesh(core_axis_name="c", subcore_axis_name="s")

    @functools.partial(
        pl.kernel,
        out_type=jax.ShapeDtypeStruct((_NC, N, C), jnp.float32),
        mesh=mesh,
        compiler_params=pltpu.CompilerParams(use_tc_tiling_on_sc=False),
        scratch_types=[
            pltpu.VMEM_SHARED((N, C), jnp.float32),
            pltpu.VMEM((TPW, B), jnp.int32),
            pltpu.VMEM((TPW, B), jnp.int32),
            pltpu.VMEM((NBUF, B, C), jnp.float32),
        ]
        + [pltpu.SemaphoreType.DMA] * (2 * NBUF),
    )
    def k(u_hbm, srcT_hbm, dstT_hbm, out_hbm, acc, sidx, didx, rows, *sems):
        gs, ss = sems[:NBUF], sems[NBUF:]
        c = lax.axis_index("c")
        s = lax.axis_index("s")
        wid = s * _NC + c
        r0 = s * RPT
        @pl.when(s < _NS - 1)
        def _():
            pltpu.sync_copy(u_hbm.at[pl.ds(r0, RPT)], acc.at[pl.ds(r0, RPT)])

        @pl.when(s == _NS - 1)
        def _():
            pltpu.sync_copy(u_hbm.at[pl.ds(r0, LAST)], acc.at[pl.ds(r0, LAST)])

        plsc.subcore_barrier()
        t0 = wid * TPW
        pltpu.sync_copy(srcT_hbm.at[pl.ds(t0, TPW)], sidx)
        pltpu.sync_copy(dstT_hbm.at[pl.ds(t0, TPW)], didx)
        for b in range(NBUF):
            pltpu.async_copy(u_hbm.at[sidx.at[b]], rows.at[b], gs[b])

        @pl.loop(0, (TPW - NBUF) // NBUF)
        def _(it):
            i = it * NBUF
            for b in range(NBUF):
                pltpu.make_async_copy(
                    u_hbm.at[sidx.at[i + b]], rows.at[b], gs[b]
                ).wait()
                pltpu.async_copy(rows.at[b], acc.at[didx.at[i + b]], ss[b], add=True)
            for b in range(NBUF):
                pltpu.make_async_copy(
                    rows.at[b], acc.at[didx.at[i + b]], ss[b]
                ).wait()
                pltpu.async_copy(u_hbm.at[sidx.at[i + NBUF + b]], rows.at[b], gs[b])

        ilast = TPW - NBUF
        for b in range(NBUF):
            pltpu.make_async_copy(
                u_hbm.at[sidx.at[ilast + b]], rows.at[b], gs[b]
            ).wait()
            pltpu.async_copy(rows.at[b], acc.at[didx.at[ilast + b]], ss[b], add=True)
        for b in range(NBUF):
            pltpu.make_async_copy(
                rows.at[b], acc.at[didx.at[ilast + b]], ss[b]
            ).wait()

        plsc.subcore_barrier()

        @pl.when(s < _NS - 1)
        def _():
            pltpu.sync_copy(acc.at[pl.ds(r0, RPT)], out_hbm.at[c, pl.ds(r0, RPT)])

        @pl.when(s == _NS - 1)
        def _():
            pltpu.sync_copy(acc.at[pl.ds(r0, LAST)], out_hbm.at[c, pl.ds(r0, LAST)])

    return k(u, srcT, dstT)


def kernel(x, edge_index, W1, Wmu, Wlv):
    N = x.shape[0]
    E = edge_index.shape[1]
    L = Wmu.shape[1]
    srcT = edge_index[0].reshape(E // _B, _B)
    dstT = edge_index[1].reshape(E // _B, _B)
    Wcat = jnp.concatenate([Wmu, Wlv], axis=1)

    hists = _deg_hists(dstT, N)
    u1, a = _enc1(hists, x, W1)
    P1 = _edge_aggregate(u1, srcT, dstT)
    u2 = _enc2(P1, u1, a, Wcat)
    P2 = _edge_aggregate(u2, srcT, dstT)
    adj, mu, logvar = _decoder(P2, u2, a, L)
    return (adj, mu, logvar)


# speedup vs baseline: 32.7772x; 1.0177x over previous
import functools

import jax
import jax.numpy as jnp
from jax import lax
from jax.experimental import pallas as pl
from jax.experimental.pallas import tpu as pltpu
from jax.experimental.pallas import tpu_sc as plsc


def _enc1(hists, x, W1, RB=1024):
    NWh, N = hists.shape
    F = x.shape[1]
    H = W1.shape[1]

    def body(h_ref, x_ref, w_ref, u_ref, a_ref):
        ones = jnp.ones((NWh, 1), jnp.float32)
        deg = lax.dot_general(h_ref[...], ones, (((0,), (0,)), ((), ()))) + 1.0
        a = lax.rsqrt(deg)
        v = jnp.dot(x_ref[...], w_ref[...], preferred_element_type=jnp.float32)
        u_ref[...] = v * a
        a_ref[...] = a

    return pl.pallas_call(
        body,
        grid=(pl.cdiv(N, RB),),
        in_specs=[
            pl.BlockSpec((NWh, RB), lambda i: (0, i)),
            pl.BlockSpec((RB, F), lambda i: (i, 0)),
            pl.BlockSpec((F, H), lambda i: (0, 0)),
        ],
        out_specs=[
            pl.BlockSpec((RB, H), lambda i: (i, 0)),
            pl.BlockSpec((RB, 1), lambda i: (i, 0)),
        ],
        out_shape=[
            jax.ShapeDtypeStruct((N, H), jnp.float32),
            jax.ShapeDtypeStruct((N, 1), jnp.float32),
        ],
    )(hists, x, W1)


def _enc2(P, u1, a, Wcat, RB=1024):
    N, H = u1.shape
    C2 = Wcat.shape[1]

    def body(p_ref, u_ref, a_ref, w_ref, o_ref):
        s = p_ref[0] + p_ref[1] - u_ref[...]
        h = jnp.maximum(s * a_ref[...], 0.0)
        o_ref[...] = (
            jnp.dot(h, w_ref[...], preferred_element_type=jnp.float32) * a_ref[...]
        )

    return pl.pallas_call(
        body,
        grid=(pl.cdiv(N, RB),),
        in_specs=[
            pl.BlockSpec((2, RB, H), lambda i: (0, i, 0)),
            pl.BlockSpec((RB, H), lambda i: (i, 0)),
            pl.BlockSpec((RB, 1), lambda i: (i, 0)),
            pl.BlockSpec((H, C2), lambda i: (0, 0)),
        ],
        out_specs=pl.BlockSpec((RB, C2), lambda i: (i, 0)),
        out_shape=jax.ShapeDtypeStruct((N, C2), jnp.float32),
    )(P, u1, a, Wcat)


def _decoder(P2, u2, a, L, BM=2048, BN=2048):
    N, C2 = u2.shape

    def body(pi_ref, ui_ref, ai_ref, pj_ref, uj_ref, aj_ref, adj_ref, mu_ref, lv_ref):
        j = pl.program_id(1)
        mi = (pi_ref[0] + pi_ref[1] - ui_ref[...]) * ai_ref[...]
        mj = (pj_ref[0] + pj_ref[1] - uj_ref[...]) * aj_ref[...]
        mui = mi[:, :L]
        muj = mj[:, :L]
        d = lax.dot_general(mui, muj, (((1,), (1,)), ((), ())))
        adj_ref[...] = jnp.tanh(d * 0.5) * 0.5 + 0.5

        @pl.when(j == 0)
        def _():
            mu_ref[...] = mui
            lv_ref[...] = mi[:, L:]

    return pl.pallas_call(
        body,
        grid=(pl.cdiv(N, BM), pl.cdiv(N, BN)),
        in_specs=[
            pl.BlockSpec((2, BM, C2), lambda i, j: (0, i, 0)),
            pl.BlockSpec((BM, C2), lambda i, j: (i, 0)),
            pl.BlockSpec((BM, 1), lambda i, j: (i, 0)),
            pl.BlockSpec((2, BN, C2), lambda i, j: (0, j, 0)),
            pl.BlockSpec((BN, C2), lambda i, j: (j, 0)),
            pl.BlockSpec((BN, 1), lambda i, j: (j, 0)),
        ],
        out_specs=[
            pl.BlockSpec((BM, BN), lambda i, j: (i, j)),
            pl.BlockSpec((BM, L), lambda i, j: (i, 0)),
            pl.BlockSpec((BM, L), lambda i, j: (i, 0)),
        ],
        out_shape=[
            jax.ShapeDtypeStruct((N, N), jnp.float32),
            jax.ShapeDtypeStruct((N, L), jnp.float32),
            jax.ShapeDtypeStruct((N, L), jnp.float32),
        ],
    )(P2, u2, a, P2, u2, a)


_NC, _NS = 2, 16
_NW = _NC * _NS


def _deg_hists(dstT, N):
    T, B = dstT.shape
    NBUF = 4
    assert T % _NW == 0
    TPW = T // _NW
    assert TPW % NBUF == 0 and B <= 128
    RPT = ((-(-N // _NS) + 7) // 8) * 8
    LAST = N - RPT * (_NS - 1)
    SBUF = ((RPT + 15) // 16) * 16
    assert 0 < LAST <= RPT and RPT % 8 == 0
    mesh = plsc.VectorSubcoreMesh(core_axis_name="c", subcore_axis_name="s")

    @functools.partial(
        pl.kernel,
        out_type=jax.ShapeDtypeStruct((_NC, N), jnp.float32),
        mesh=mesh,
        compiler_params=pltpu.CompilerParams(use_tc_tiling_on_sc=False),
        scratch_types=[
            pltpu.VMEM_SHARED((N,), jnp.float32),
            pltpu.VMEM((TPW, B), jnp.int32),
            pltpu.VMEM((128,), jnp.float32),
            pltpu.VMEM((SBUF,), jnp.float32),
        ]
        + [pltpu.SemaphoreType.DMA] * NBUF,
    )
    def k(dstT_hbm, out_hbm, hist, didx, onez, stage, *ss):
        c = lax.axis_index("c")
        s = lax.axis_index("s")
        wid = s * _NC + c
        r0 = s * RPT
        ones16 = jnp.ones((16,), jnp.float32)
        zeros16 = jnp.zeros((16,), jnp.float32)
        for i in range(8):
            onez[pl.ds(16 * i, 16)] = ones16

        @pl.loop(0, SBUF // 16)
        def _(i):
            stage[pl.ds(16 * i, 16)] = zeros16

        @pl.when(s < _NS - 1)
        def _():
            pltpu.sync_copy(stage.at[pl.ds(0, RPT)], hist.at[pl.ds(r0, RPT)])

        @pl.when(s == _NS - 1)
        def _():
            pltpu.sync_copy(stage.at[pl.ds(0, LAST)], hist.at[pl.ds(r0, LAST)])

        plsc.subcore_barrier()
        t0 = wid * TPW
        pltpu.sync_copy(dstT_hbm.at[pl.ds(t0, TPW)], didx)

        @pl.loop(0, TPW // NBUF)
        def _(it):
            i = it * NBUF
            for b in range(NBUF):
                pltpu.async_copy(
                    onez.at[pl.ds(0, B)], hist.at[didx.at[i + b]], ss[b], add=True
                )
            for b in range(NBUF):
                pltpu.make_async_copy(
                    onez.at[pl.ds(0, B)], hist.at[didx.at[i + b]], ss[b]
                ).wait()

        plsc.subcore_barrier()

        @pl.when(s < _NS - 1)
        def _():
            pltpu.sync_copy(hist.at[pl.ds(r0, RPT)], stage.at[pl.ds(0, RPT)])
            pltpu.sync_copy(stage.at[pl.ds(0, RPT)], out_hbm.at[c, pl.ds(r0, RPT)])

        @pl.when(s == _NS - 1)
        def _():
            pltpu.sync_copy(hist.at[pl.ds(r0, LAST)], stage.at[pl.ds(0, LAST)])
            pltpu.sync_copy(stage.at[pl.ds(0, LAST)], out_hbm.at[c, pl.ds(r0, LAST)])

    return k(dstT)


_B = 125


def _edge_aggregate(u, srcT, dstT):
    N, C = u.shape
    T, B = srcT.shape
    NBUF = 4
    assert T % _NW == 0
    TPW = T // _NW
    assert TPW % NBUF == 0 and TPW >= 2 * NBUF and B <= 128
    RPT = ((-(-N // _NS) + 7) // 8) * 8
    LAST = N - RPT * (_NS - 1)
    assert 0 < LAST <= RPT and RPT % 8 == 0
    mesh = plsc.VectorSubcoreMesh(core_axis_name="c", subcore_axis_name="s")

    @functools.partial(
        pl.kernel,
        out_type=jax.ShapeDtypeStruct((_NC, N, C), jnp.float32),
        mesh=mesh,
        compiler_params=pltpu.CompilerParams(use_tc_tiling_on_sc=False),
        scratch_types=[
            pltpu.VMEM_SHARED((N, C), jnp.float32),
            pltpu.VMEM((TPW, B), jnp.int32),
            pltpu.VMEM((TPW, B), jnp.int32),
            pltpu.VMEM((NBUF, B, C), jnp.float32),
        ]
        + [pltpu.SemaphoreType.DMA] * (2 * NBUF),
    )
    def k(u_hbm, srcT_hbm, dstT_hbm, out_hbm, acc, sidx, didx, rows, *sems):
        gs, ss = sems[:NBUF], sems[NBUF:]
        c = lax.axis_index("c")
        s = lax.axis_index("s")
        wid = s * _NC + c
        r0 = s * RPT
        @pl.when(s < _NS - 1)
        def _():
            pltpu.sync_copy(u_hbm.at[pl.ds(r0, RPT)], acc.at[pl.ds(r0, RPT)])

        @pl.when(s == _NS - 1)
        def _():
            pltpu.sync_copy(u_hbm.at[pl.ds(r0, LAST)], acc.at[pl.ds(r0, LAST)])

        plsc.subcore_barrier()
        t0 = wid * TPW
        pltpu.sync_copy(srcT_hbm.at[pl.ds(t0, TPW)], sidx)
        pltpu.sync_copy(dstT_hbm.at[pl.ds(t0, TPW)], didx)
        for b in range(NBUF):
            pltpu.async_copy(u_hbm.at[sidx.at[b]], rows.at[b], gs[b])

        @pl.loop(0, (TPW - NBUF) // NBUF)
        def _(it):
            i = it * NBUF
            for b in range(NBUF):
                pltpu.make_async_copy(
                    u_hbm.at[sidx.at[i + b]], rows.at[b], gs[b]
                ).wait()
                pltpu.async_copy(rows.at[b], acc.at[didx.at[i + b]], ss[b], add=True)
            for b in range(NBUF):
                pltpu.make_async_copy(
                    rows.at[b], acc.at[didx.at[i + b]], ss[b]
                ).wait()
                pltpu.async_copy(u_hbm.at[sidx.at[i + NBUF + b]], rows.at[b], gs[b])

        ilast = TPW - NBUF
        for b in range(NBUF):
            pltpu.make_async_copy(
                u_hbm.at[sidx.at[ilast + b]], rows.at[b], gs[b]
            ).wait()
            pltpu.async_copy(rows.at[b], acc.at[didx.at[ilast + b]], ss[b], add=True)
        for b in range(NBUF):
            pltpu.make_async_copy(
                rows.at[b], acc.at[didx.at[ilast + b]], ss[b]
            ).wait()

        plsc.subcore_barrier()

        @pl.when(s < _NS - 1)
        def _():
            pltpu.sync_copy(acc.at[pl.ds(r0, RPT)], out_hbm.at[c, pl.ds(r0, RPT)])

        @pl.when(s == _NS - 1)
        def _():
            pltpu.sync_copy(acc.at[pl.ds(r0, LAST)], out_hbm.at[c, pl.ds(r0, LAST)])

    return k(u, srcT, dstT)


def kernel(x, edge_index, W1, Wmu, Wlv):
    N = x.shape[0]
    E = edge_index.shape[1]
    L = Wmu.shape[1]
    srcT = edge_index[0].reshape(E // _B, _B)
    dstT = edge_index[1].reshape(E // _B, _B)
    Wcat = jnp.concatenate([Wmu, Wlv], axis=1)

    hists = _deg_hists(dstT, N)
    u1, a = _enc1(hists, x, W1)
    P1 = _edge_aggregate(u1, srcT, dstT)
    u2 = _enc2(P1, u1, a, Wcat)
    P2 = _edge_aggregate(u2, srcT, dstT)
    adj, mu, logvar = _decoder(P2, u2, a, L)
    return (adj, mu, logvar)


# speedup vs baseline: 33.4310x; 1.0199x over previous
import functools

import jax
import jax.numpy as jnp
from jax import lax
from jax.experimental import pallas as pl
from jax.experimental.pallas import tpu as pltpu
from jax.experimental.pallas import tpu_sc as plsc


def _enc1(hists, x, W1, RB=1024):
    NWh, N = hists.shape
    F = x.shape[1]
    H = W1.shape[1]

    def body(h_ref, x_ref, w_ref, u_ref, a_ref):
        ones = jnp.ones((NWh, 1), jnp.float32)
        deg = lax.dot_general(h_ref[...], ones, (((0,), (0,)), ((), ()))) + 1.0
        a = lax.rsqrt(deg)
        v = jnp.dot(x_ref[...], w_ref[...], preferred_element_type=jnp.float32)
        u_ref[...] = v * a
        a_ref[...] = a

    return pl.pallas_call(
        body,
        grid=(pl.cdiv(N, RB),),
        in_specs=[
            pl.BlockSpec((NWh, RB), lambda i: (0, i)),
            pl.BlockSpec((RB, F), lambda i: (i, 0)),
            pl.BlockSpec((F, H), lambda i: (0, 0)),
        ],
        out_specs=[
            pl.BlockSpec((RB, H), lambda i: (i, 0)),
            pl.BlockSpec((RB, 1), lambda i: (i, 0)),
        ],
        out_shape=[
            jax.ShapeDtypeStruct((N, H), jnp.float32),
            jax.ShapeDtypeStruct((N, 1), jnp.float32),
        ],
    )(hists, x, W1)


def _enc2(P, u1, a, Wcat, RB=1024):
    N, H = u1.shape
    C2 = Wcat.shape[1]

    def body(p_ref, u_ref, a_ref, w_ref, o_ref):
        s = p_ref[0] + p_ref[1] - u_ref[...]
        h = jnp.maximum(s * a_ref[...], 0.0)
        o_ref[...] = (
            jnp.dot(h, w_ref[...], preferred_element_type=jnp.float32) * a_ref[...]
        )

    return pl.pallas_call(
        body,
        grid=(pl.cdiv(N, RB),),
        in_specs=[
            pl.BlockSpec((2, RB, H), lambda i: (0, i, 0)),
            pl.BlockSpec((RB, H), lambda i: (i, 0)),
            pl.BlockSpec((RB, 1), lambda i: (i, 0)),
            pl.BlockSpec((H, C2), lambda i: (0, 0)),
        ],
        out_specs=pl.BlockSpec((RB, C2), lambda i: (i, 0)),
        out_shape=jax.ShapeDtypeStruct((N, C2), jnp.float32),
    )(P, u1, a, Wcat)


def _decoder(P2, u2, a, L, BM=2048, BN=2048):
    N, C2 = u2.shape

    def body(pi_ref, ui_ref, ai_ref, pj_ref, uj_ref, aj_ref, adj_ref, mu_ref, lv_ref):
        j = pl.program_id(1)
        mi = (pi_ref[0] + pi_ref[1] - ui_ref[...]) * ai_ref[...]
        mj = (pj_ref[0] + pj_ref[1] - uj_ref[...]) * aj_ref[...]
        mui = mi[:, :L]
        muj = mj[:, :L]
        d = lax.dot_general(mui, muj, (((1,), (1,)), ((), ())))
        adj_ref[...] = jnp.tanh(d * 0.5) * 0.5 + 0.5

        @pl.when(j == 0)
        def _():
            mu_ref[...] = mui
            lv_ref[...] = mi[:, L:]

    return pl.pallas_call(
        body,
        grid=(pl.cdiv(N, BM), pl.cdiv(N, BN)),
        in_specs=[
            pl.BlockSpec((2, BM, C2), lambda i, j: (0, i, 0)),
            pl.BlockSpec((BM, C2), lambda i, j: (i, 0)),
            pl.BlockSpec((BM, 1), lambda i, j: (i, 0)),
            pl.BlockSpec((2, BN, C2), lambda i, j: (0, j, 0)),
            pl.BlockSpec((BN, C2), lambda i, j: (j, 0)),
            pl.BlockSpec((BN, 1), lambda i, j: (j, 0)),
        ],
        out_specs=[
            pl.BlockSpec((BM, BN), lambda i, j: (i, j)),
            pl.BlockSpec((BM, L), lambda i, j: (i, 0)),
            pl.BlockSpec((BM, L), lambda i, j: (i, 0)),
        ],
        out_shape=[
            jax.ShapeDtypeStruct((N, N), jnp.float32),
            jax.ShapeDtypeStruct((N, L), jnp.float32),
            jax.ShapeDtypeStruct((N, L), jnp.float32),
        ],
    )(P2, u2, a, P2, u2, a)


_NC, _NS = 2, 16
_NW = _NC * _NS


def _deg_hists(dstT, N):
    T, B = dstT.shape
    NBUF = 4
    assert T % _NW == 0
    TPW = T // _NW
    assert TPW % NBUF == 0 and B <= 128
    RPT = ((-(-N // _NS) + 7) // 8) * 8
    LAST = N - RPT * (_NS - 1)
    SBUF = ((RPT + 15) // 16) * 16
    assert 0 < LAST <= RPT and RPT % 8 == 0
    mesh = plsc.VectorSubcoreMesh(core_axis_name="c", subcore_axis_name="s")

    @functools.partial(
        pl.kernel,
        out_type=jax.ShapeDtypeStruct((_NC, N), jnp.float32),
        mesh=mesh,
        compiler_params=pltpu.CompilerParams(use_tc_tiling_on_sc=False),
        scratch_types=[
            pltpu.VMEM_SHARED((N,), jnp.float32),
            pltpu.VMEM((TPW, B), jnp.int32),
            pltpu.VMEM((128,), jnp.float32),
            pltpu.VMEM((SBUF,), jnp.float32),
        ]
        + [pltpu.SemaphoreType.DMA] * NBUF,
    )
    def k(dstT_hbm, out_hbm, hist, didx, onez, stage, *ss):
        c = lax.axis_index("c")
        s = lax.axis_index("s")
        wid = s * _NC + c
        r0 = s * RPT
        ones16 = jnp.ones((16,), jnp.float32)
        zeros16 = jnp.zeros((16,), jnp.float32)
        for i in range(8):
            onez[pl.ds(16 * i, 16)] = ones16

        @pl.loop(0, SBUF // 16)
        def _(i):
            stage[pl.ds(16 * i, 16)] = zeros16

        @pl.when(s < _NS - 1)
        def _():
            pltpu.sync_copy(stage.at[pl.ds(0, RPT)], hist.at[pl.ds(r0, RPT)])

        @pl.when(s == _NS - 1)
        def _():
            pltpu.sync_copy(stage.at[pl.ds(0, LAST)], hist.at[pl.ds(r0, LAST)])

        plsc.subcore_barrier()
        t0 = wid * TPW
        pltpu.sync_copy(dstT_hbm.at[pl.ds(t0, TPW)], didx)

        @pl.loop(0, TPW // NBUF)
        def _(it):
            i = it * NBUF
            for b in range(NBUF):
                pltpu.async_copy(
                    onez.at[pl.ds(0, B)], hist.at[didx.at[i + b]], ss[b], add=True
                )
            for b in range(NBUF):
                pltpu.make_async_copy(
                    onez.at[pl.ds(0, B)], hist.at[didx.at[i + b]], ss[b]
                ).wait()

        plsc.subcore_barrier()

        @pl.when(s < _NS - 1)
        def _():
            pltpu.sync_copy(hist.at[pl.ds(r0, RPT)], stage.at[pl.ds(0, RPT)])
            pltpu.sync_copy(stage.at[pl.ds(0, RPT)], out_hbm.at[c, pl.ds(r0, RPT)])

        @pl.when(s == _NS - 1)
        def _():
            pltpu.sync_copy(hist.at[pl.ds(r0, LAST)], stage.at[pl.ds(0, LAST)])
            pltpu.sync_copy(stage.at[pl.ds(0, LAST)], out_hbm.at[c, pl.ds(r0, LAST)])

    return k(dstT)


_B = 125


def _edge_aggregate(u, srcT, dstT):
    N, C = u.shape
    T, B = srcT.shape
    NBUF = 8
    assert T % _NW == 0
    TPW = T // _NW
    assert TPW % NBUF == 0 and TPW >= 2 * NBUF and B <= 128
    RPT = ((-(-N // _NS) + 7) // 8) * 8
    LAST = N - RPT * (_NS - 1)
    assert 0 < LAST <= RPT and RPT % 8 == 0
    mesh = plsc.VectorSubcoreMesh(core_axis_name="c", subcore_axis_name="s")

    @functools.partial(
        pl.kernel,
        out_type=jax.ShapeDtypeStruct((_NC, N, C), jnp.float32),
        mesh=mesh,
        compiler_params=pltpu.CompilerParams(use_tc_tiling_on_sc=False),
        scratch_types=[
            pltpu.VMEM_SHARED((N, C), jnp.float32),
            pltpu.VMEM((TPW, B), jnp.int32),
            pltpu.VMEM((TPW, B), jnp.int32),
            pltpu.VMEM((NBUF, B, C), jnp.float32),
        ]
        + [pltpu.SemaphoreType.DMA] * (2 * NBUF),
    )
    def k(u_hbm, srcT_hbm, dstT_hbm, out_hbm, acc, sidx, didx, rows, *sems):
        gs, ss = sems[:NBUF], sems[NBUF:]
        c = lax.axis_index("c")
        s = lax.axis_index("s")
        wid = s * _NC + c
        r0 = s * RPT
        @pl.when(s < _NS - 1)
        def _():
            pltpu.sync_copy(u_hbm.at[pl.ds(r0, RPT)], acc.at[pl.ds(r0, RPT)])

        @pl.when(s == _NS - 1)
        def _():
            pltpu.sync_copy(u_hbm.at[pl.ds(r0, LAST)], acc.at[pl.ds(r0, LAST)])

        plsc.subcore_barrier()
        t0 = wid * TPW
        pltpu.sync_copy(srcT_hbm.at[pl.ds(t0, TPW)], sidx)
        pltpu.sync_copy(dstT_hbm.at[pl.ds(t0, TPW)], didx)
        for b in range(NBUF):
            pltpu.async_copy(u_hbm.at[sidx.at[b]], rows.at[b], gs[b])

        @pl.loop(0, (TPW - NBUF) // NBUF)
        def _(it):
            i = it * NBUF
            for b in range(NBUF):
                pltpu.make_async_copy(
                    u_hbm.at[sidx.at[i + b]], rows.at[b], gs[b]
                ).wait()
                pltpu.async_copy(rows.at[b], acc.at[didx.at[i + b]], ss[b], add=True)
            for b in range(NBUF):
                pltpu.make_async_copy(
                    rows.at[b], acc.at[didx.at[i + b]], ss[b]
                ).wait()
                pltpu.async_copy(u_hbm.at[sidx.at[i + NBUF + b]], rows.at[b], gs[b])

        ilast = TPW - NBUF
        for b in range(NBUF):
            pltpu.make_async_copy(
                u_hbm.at[sidx.at[ilast + b]], rows.at[b], gs[b]
            ).wait()
            pltpu.async_copy(rows.at[b], acc.at[didx.at[ilast + b]], ss[b], add=True)
        for b in range(NBUF):
            pltpu.make_async_copy(
                rows.at[b], acc.at[didx.at[ilast + b]], ss[b]
            ).wait()

        plsc.subcore_barrier()

        @pl.when(s < _NS - 1)
        def _():
            pltpu.sync_copy(acc.at[pl.ds(r0, RPT)], out_hbm.at[c, pl.ds(r0, RPT)])

        @pl.when(s == _NS - 1)
        def _():
            pltpu.sync_copy(acc.at[pl.ds(r0, LAST)], out_hbm.at[c, pl.ds(r0, LAST)])

    return k(u, srcT, dstT)


def kernel(x, edge_index, W1, Wmu, Wlv):
    N = x.shape[0]
    E = edge_index.shape[1]
    L = Wmu.shape[1]
    srcT = edge_index[0].reshape(E // _B, _B)
    dstT = edge_index[1].reshape(E // _B, _B)
    Wcat = jnp.concatenate([Wmu, Wlv], axis=1)

    hists = _deg_hists(dstT, N)
    u1, a = _enc1(hists, x, W1)
    P1 = _edge_aggregate(u1, srcT, dstT)
    u2 = _enc2(P1, u1, a, Wcat)
    P2 = _edge_aggregate(u2, srcT, dstT)
    adj, mu, logvar = _decoder(P2, u2, a, L)
    return (adj, mu, logvar)
